# baseline (speedup 1.0000x reference)
import functools

import jax
import jax.numpy as jnp
from jax import lax
from jax.experimental import pallas as pl
from jax.experimental.pallas import tpu as pltpu
from jax.experimental.pallas import tpu_sc as plsc

D_MODEL = 64
N_VOCAB = 1000
B = 4096
T = 200

NC = 2
NS = 16
NW = NC * NS

TOTAL = B * T
LANES = 16
BTILE = 128
DTILE = 8
NBT = B // BTILE
NDT = D_MODEL // DTILE
QUART = B // 4
UNITS = T * 4
PER_W = UNITS // NW
UNIT_IDX = QUART
NGRP = QUART // LANES
CHUNK = DTILE * 8 * BTILE
T_STRIDE = NDT * NBT * DTILE * BTILE
DT_STRIDE = NBT * DTILE * BTILE


def _bias_body(table_ref, base_ref, out_ref):
    out_ref[...] = table_ref[...] + base_ref[...]


def _bias_table(emb_table, base_emb):
    return pl.pallas_call(
        _bias_body,
        out_shape=jax.ShapeDtypeStruct(emb_table.shape, emb_table.dtype),
    )(emb_table, base_emb.reshape(1, D_MODEL))


def _gather_body(table_hbm, idx_hbm, out_hbm, table_v, idx_v, outb, ssem):
    wid = lax.axis_index("s") * NC + lax.axis_index("c")
    pltpu.sync_copy(table_hbm, table_v)
    u0 = wid * PER_W
    pltpu.sync_copy(idx_hbm.at[pl.ds(u0 * UNIT_IDX, PER_W * UNIT_IDX)], idx_v)

    def drain_store(sb):
        pltpu.make_async_copy(
            out_hbm.at[pl.ds(0, CHUNK)], outb.at[sb], ssem.at[sb]
        ).wait()

    def unit(j, _):
        t = (u0 + j) // 4
        q = (u0 + j) % 4
        out_base = t * T_STRIDE + q * CHUNK
        for dt in range(NDT):
            sb = dt % 2

            if dt >= 2:
                drain_store(sb)
            else:

                @pl.when(j >= 1)
                def _():
                    drain_store(sb)

            def grp(g, _, dt=dt, sb=sb):
                idxvec = idx_v[pl.ds(j * UNIT_IDX + g * LANES, LANES)]
                rowoff = idxvec * D_MODEL
                off = (
                    lax.shift_right_logical(g, 3) * (DTILE * BTILE)
                    + lax.bitwise_and(g, 7) * LANES
                )
                for ds in range(DTILE):
                    d = dt * DTILE + ds
                    v = plsc.load_gather(table_v, [rowoff + d])
                    outb[sb, pl.ds(off + ds * BTILE, LANES)] = v
                return ()

            lax.fori_loop(0, NGRP, grp, (), unroll=False)
            pltpu.async_copy(
                outb.at[sb],
                out_hbm.at[pl.ds(out_base + dt * DT_STRIDE, CHUNK)],
                ssem.at[sb],
            )
        return ()

    lax.fori_loop(0, PER_W, unit, (), unroll=False)
    drain_store(0)
    drain_store(1)


@jax.jit
def kernel(actions, emb_table, base_emb):
    biased = _bias_table(emb_table, base_emb)
    idx = actions.astype(jnp.int32).T.reshape(TOTAL)

    mesh = plsc.VectorSubcoreMesh(core_axis_name="c", subcore_axis_name="s")
    out = pl.kernel(
        _gather_body,
        out_type=jax.ShapeDtypeStruct((TOTAL * D_MODEL,), jnp.float32),
        mesh=mesh,
        scratch_types=[
            pltpu.VMEM((N_VOCAB * D_MODEL,), jnp.float32),
            pltpu.VMEM((PER_W * UNIT_IDX,), jnp.int32),
            pltpu.VMEM((2, CHUNK), jnp.float32),
            pltpu.SemaphoreType.DMA((2,)),
        ],
        compiler_params=pltpu.CompilerParams(
            use_tc_tiling_on_sc=False, needs_layout_passes=False
        ),
    )(biased.reshape(N_VOCAB * D_MODEL), idx)
    out5 = out.reshape(T, NDT, NBT, DTILE, BTILE)
    out3 = out5.transpose(2, 4, 0, 1, 3).reshape(B, T, D_MODEL)
    return out3[:, :, None, :]
